# Initial kernel scaffold; baseline (speedup 1.0000x reference)
import jax
import jax.numpy as jnp
from jax.experimental import pallas as pl


def kernel(x, mask_sparse, W, bias, W_phi):
    raise NotImplementedError("write your pallas kernel here")


# speedup vs baseline: 1.2148x; 1.2148x over previous
import functools

import jax
import jax.numpy as jnp
from jax import lax
from jax.experimental import pallas as pl
from jax.experimental.pallas import tpu as pltpu
from jax.experimental.pallas import tpu_sc as plsc

N = 10000
E = 160000
D = 256
NUM_ITERS = 8
EPS = 0.1
GAMMA = 0.1

NW = 32
R = 320
NPAD = NW * R
K = 64
EPAD = E + NW * K
ZROW = N

BR = 1024


def _mm_body(x_ref, w_ref, deg_ref, z1_ref, y_ref):
    z = jnp.dot(x_ref[...], w_ref[...], preferred_element_type=jnp.float32)
    deg = deg_ref[...]
    dinv = jnp.where(deg > 0.0, lax.rsqrt(deg), 0.0)
    z1_ref[...] = z[:, :D]
    y_ref[...] = z[:, D:] * dinv


def _tc_matmul(x, wcat, deg):
    return pl.pallas_call(
        _mm_body,
        grid=(NPAD // BR,),
        in_specs=[
            pl.BlockSpec((BR, D), lambda i: (i, 0)),
            pl.BlockSpec((D, 2 * D), lambda i: (0, 0)),
            pl.BlockSpec((BR, 1), lambda i: (i, 0)),
        ],
        out_specs=[
            pl.BlockSpec((BR, D), lambda i: (i, 0)),
            pl.BlockSpec((BR, D), lambda i: (i, 0)),
        ],
        out_shape=[
            jax.ShapeDtypeStruct((NPAD, D), jnp.float32),
            jax.ShapeDtypeStruct((NPAD, D), jnp.float32),
        ],
    )(x, wcat, deg)


def _upd_body_first(z1_ref, s_ref, deg_ref, b_ref, x_ref, xo_ref, mo_ref):
    deg = deg_ref[...]
    dinv = jnp.where(deg > 0.0, lax.rsqrt(deg), 0.0)
    g = dinv * s_ref[...]
    h = jnp.tanh(z1_ref[...] + g + b_ref[...])
    xn = x_ref[...] + EPS * h
    xo_ref[...] = xn
    mo_ref[...] = xn


def _upd_body(z1_ref, s_ref, deg_ref, b_ref, x_ref, m_ref, xo_ref, mo_ref):
    deg = deg_ref[...]
    dinv = jnp.where(deg > 0.0, lax.rsqrt(deg), 0.0)
    g = dinv * s_ref[...]
    h = jnp.tanh(z1_ref[...] + g + b_ref[...])
    xn = x_ref[...] + EPS * h
    xo_ref[...] = xn
    mo_ref[...] = jnp.maximum(m_ref[...], xn)


def _tc_update(z1, s, deg, bias2d, x, m):
    row = pl.BlockSpec((BR, D), lambda i: (i, 0))
    specs = [row, row,
             pl.BlockSpec((BR, 1), lambda i: (i, 0)),
             pl.BlockSpec((1, D), lambda i: (0, 0)),
             row]
    args = [z1, s, deg, bias2d, x]
    body = _upd_body_first
    if m is not None:
        specs.append(row)
        args.append(m)
        body = _upd_body
    return pl.pallas_call(
        body,
        grid=(NPAD // BR,),
        in_specs=specs,
        out_specs=[row, row],
        out_shape=[
            jax.ShapeDtypeStruct((NPAD, D), jnp.float32),
            jax.ShapeDtypeStruct((NPAD, D), jnp.float32),
        ],
    )(*args)


@functools.cache
def _sc_kernels():
    mesh = plsc.VectorSubcoreMesh(core_axis_name="c", subcore_axis_name="s")


    @functools.partial(
        pl.kernel,
        mesh=mesh,
        out_type=jax.ShapeDtypeStruct((NPAD * D,), jnp.float32),
        scratch_types=[
            pltpu.VMEM((R * D,), jnp.float32),
            pltpu.VMEM((K,), jnp.int32),
            pltpu.VMEM((K + 16,), jnp.int32),
            pltpu.VMEM((K, D), jnp.float32),
            pltpu.VMEM((NW + 16,), jnp.int32),
            pltpu.VMEM((NW + 16,), jnp.int32),
            pltpu.SemaphoreType.DMA,
        ],
    )
    def sc_scatter(y_hbm, y1_hbm, srcp_hbm, dstp_hbm, cnt_hbm, off_hbm, s1_hbm,
                   accf, srcv, dstv, rows, cntv, offv, sem):
        wid = lax.axis_index("s") * 2 + lax.axis_index("c")
        base = wid * R
        pltpu.sync_copy(cnt_hbm, cntv)
        pltpu.sync_copy(off_hbm, offv)
        pltpu.sync_copy(y1_hbm.at[pl.ds(base * D, R * D)], accf)
        nchunks = cntv[pl.ds(wid, 16)][0]
        coff = offv[pl.ds(wid, 16)][0]

        def chunk_body(c, carry):
            e0 = (coff + c) * K
            pltpu.sync_copy(srcp_hbm.at[pl.ds(e0, K)], srcv)
            pltpu.sync_copy(dstp_hbm.at[pl.ds(e0, K)], dstv.at[pl.ds(0, K)])
            pltpu.async_copy(y_hbm.at[srcv], rows, sem).wait()
            for j in range(K):
                dl = dstv[pl.ds(j, 16)][0]
                dbase = dl * D
                for t in range(D // 16):
                    v = rows[j, pl.ds(t * 16, 16)]
                    plsc.addupdate(accf.at[pl.ds(dbase + t * 16, 16)], v)
            return carry

        lax.fori_loop(0, nchunks, chunk_body, 0)
        pltpu.sync_copy(accf, s1_hbm.at[pl.ds(base * D, R * D)])

    return sc_scatter


def _route_edges(src, dst):
    order = jnp.argsort(dst)
    srcs = src[order]
    dsts = dst[order]
    tsort = dsts // R
    bounds = jnp.searchsorted(dsts, jnp.arange(NW + 1, dtype=jnp.int32) * R)
    counts = bounds[1:] - bounds[:-1]
    coff = bounds[:-1]
    pc = ((counts + K - 1) // K) * K
    poff = jnp.concatenate([jnp.zeros((1,), pc.dtype), jnp.cumsum(pc)[:-1]])
    pos = poff[tsort] + jnp.arange(E, dtype=jnp.int32) - coff[tsort]
    srcp = jnp.full((EPAD,), ZROW, jnp.int32).at[pos].set(srcs)
    dstp = jnp.zeros((EPAD,), jnp.int32).at[pos].set(dsts - tsort * R)
    cnt = jnp.pad((pc // K).astype(jnp.int32), (0, 16))
    off = jnp.pad((poff // K).astype(jnp.int32), (0, 16))
    nb = jnp.searchsorted(dsts, jnp.arange(NPAD + 1, dtype=jnp.int32))
    deg_edges = (nb[1:] - nb[:-1]).astype(jnp.float32)
    return srcp, dstp, cnt, off, deg_edges


def kernel(x, mask_sparse, W, bias, W_phi):
    src = mask_sparse[0]
    dst = mask_sparse[1]

    srcp, dstp, cnt, off, deg_edges = _route_edges(src, dst)
    selfw = jnp.where(jnp.arange(NPAD) < N, 1.0, 0.0)
    deg = (deg_edges + selfw).reshape(NPAD, 1)

    awt = W.T - W - GAMMA * jnp.eye(D, dtype=W.dtype)
    wcat = jnp.concatenate([awt, W_phi], axis=1)
    bias2d = bias.reshape(1, D)

    xp = jnp.pad(x, ((0, NPAD - N), (0, 0)))

    sc_scatter = _sc_kernels()

    m = None
    for _ in range(NUM_ITERS):
        z1, y = _tc_matmul(xp, wcat, deg)
        s = sc_scatter(y, y.reshape(NPAD * D), srcp, dstp, cnt, off)
        s = s.reshape(NPAD, D)
        xp, m = _tc_update(z1, s, deg, bias2d, xp, m)

    return m[:N]
